# Initial kernel scaffold; baseline (speedup 1.0000x reference)
import jax
import jax.numpy as jnp
from jax.experimental import pallas as pl


def kernel(x, edge_index, batch_index, W1, b1, W2, b2, W_out, b_out):
    raise NotImplementedError("write your pallas kernel here")


# speedup vs baseline: 12.9753x; 12.9753x over previous
import functools

import jax
import jax.numpy as jnp
from jax import lax
from jax.experimental import pallas as pl
from jax.experimental.pallas import tpu as pltpu
from jax.experimental.pallas import tpu_sc as plsc

NC = 2
NS = 16
NW = NC * NS

N = 10000
E = 320000
D = 128
G = 64

_F32 = jnp.float32


def _make_deg():
    EPT = E // NW
    K = 80
    NCHUNK = EPT // K
    CO = 640
    CO_LAST = N - CO * (NS - 1)
    ZR = 80
    mesh = plsc.VectorSubcoreMesh(core_axis_name="c", subcore_axis_name="s")

    @functools.partial(
        pl.kernel,
        out_type=jax.ShapeDtypeStruct((NC, N, 16), _F32),
        mesh=mesh,
        scratch_types=[
            pltpu.VMEM((K,), jnp.int32),
            pltpu.VMEM((K, 16), _F32),
            pltpu.VMEM((ZR, 16), _F32),
            pltpu.VMEM_SHARED((N, 16), _F32),
        ],
    )
    def deg_kernel(dst_hbm, out_hbm, idx_v, ones_v, zero_v, deg_sh):
        c = lax.axis_index("c")
        s = lax.axis_index("s")
        w = c * NS + s
        ones16 = jnp.ones((16,), _F32)
        zero16 = jnp.zeros((16,), _F32)

        def init_ones(i, _):
            ones_v[i, :] = ones16
            return 0

        lax.fori_loop(0, K, init_ones, 0)

        def init_zero(i, _):
            zero_v[i, :] = zero16
            return 0

        lax.fori_loop(0, ZR, init_zero, 0)
        r0 = pl.multiple_of(s * CO, 8)
        nz = jnp.where(s == NS - 1, CO_LAST // ZR, CO // ZR)

        def zbody(k, _):
            pltpu.sync_copy(
                zero_v, deg_sh.at[pl.ds(pl.multiple_of(r0 + k * ZR, 8), ZR), :])
            return 0

        lax.fori_loop(0, nz, zbody, 0)
        plsc.subcore_barrier()

        e0 = w * EPT

        def body(g, _):
            pltpu.sync_copy(
                dst_hbm.at[pl.ds(pl.multiple_of(e0 + g * K, 8), K)], idx_v)
            pltpu.sync_copy(ones_v, deg_sh.at[idx_v], add=True)
            return 0

        lax.fori_loop(0, NCHUNK, body, 0)
        plsc.subcore_barrier()

        @pl.when(s < NS - 1)
        def _():
            pltpu.sync_copy(deg_sh.at[pl.ds(r0, CO), :],
                            out_hbm.at[c, pl.ds(r0, CO), :])

        @pl.when(s == NS - 1)
        def _():
            r1 = pl.multiple_of((NS - 1) * CO, 8)
            pltpu.sync_copy(deg_sh.at[pl.ds(r1, CO_LAST), :],
                            out_hbm.at[c, pl.ds(r1, CO_LAST), :])

    return deg_kernel


_deg = _make_deg()


def _make_msgpass():
    EPT = E // NW
    K = 80
    NCHUNK = EPT // K
    CO = 640
    CO_LAST = N - CO * (NS - 1)
    ZR = 80
    mesh = plsc.VectorSubcoreMesh(core_axis_name="c", subcore_axis_name="s")

    @functools.partial(
        pl.kernel,
        out_type=jax.ShapeDtypeStruct((NC, N, D), _F32),
        mesh=mesh,
        scratch_types=[
            pltpu.VMEM((K,), jnp.int32),
            pltpu.VMEM((K,), jnp.int32),
            pltpu.VMEM((K, D), _F32),
            pltpu.VMEM((ZR, D), _F32),
            pltpu.VMEM_SHARED((N, D), _F32),
            pltpu.SemaphoreType.DMA,
        ],
    )
    def msgpass(hp_hbm, src_hbm, dst_hbm, out_hbm,
                src_v, dst_v, rows_v, zero_v, acc_sh, sem):
        c = lax.axis_index("c")
        s = lax.axis_index("s")
        w = c * NS + s
        zero16 = jnp.zeros((16,), _F32)

        def zinit(i, _):
            for cc in range(D // 16):
                zero_v[i, pl.ds(cc * 16, 16)] = zero16
            return 0

        lax.fori_loop(0, ZR, zinit, 0)
        r0 = pl.multiple_of(s * CO, 8)
        nz = jnp.where(s == NS - 1, CO_LAST // ZR, CO // ZR)

        def zbody(k, _):
            pltpu.sync_copy(
                zero_v, acc_sh.at[pl.ds(pl.multiple_of(r0 + k * ZR, 8), ZR), :])
            return 0

        lax.fori_loop(0, nz, zbody, 0)
        plsc.subcore_barrier()

        e0 = w * EPT

        def body(g, _):
            base = pl.multiple_of(e0 + g * K, 8)
            pltpu.sync_copy(src_hbm.at[pl.ds(base, K)], src_v)
            pltpu.sync_copy(dst_hbm.at[pl.ds(base, K)], dst_v)
            pltpu.async_copy(hp_hbm.at[src_v], rows_v, sem).wait()
            pltpu.sync_copy(rows_v, acc_sh.at[dst_v], add=True)
            return 0

        lax.fori_loop(0, NCHUNK, body, 0)
        plsc.subcore_barrier()

        @pl.when(s < NS - 1)
        def _():
            pltpu.sync_copy(acc_sh.at[pl.ds(r0, CO), :],
                            out_hbm.at[c, pl.ds(r0, CO), :])

        @pl.when(s == NS - 1)
        def _():
            r1 = pl.multiple_of((NS - 1) * CO, 8)
            pltpu.sync_copy(acc_sh.at[pl.ds(r1, CO_LAST), :],
                            out_hbm.at[c, pl.ds(r1, CO_LAST), :])

    return msgpass


_msgpass = _make_msgpass()


def _make_pool():
    RPW = 320
    LAST = N - RPW * (NW - 1)
    GP = 8
    mesh = plsc.VectorSubcoreMesh(core_axis_name="c", subcore_axis_name="s")

    @functools.partial(
        pl.kernel,
        out_type=(jax.ShapeDtypeStruct((NC, G, D), _F32),
                  jax.ShapeDtypeStruct((NC, G, D), _F32),
                  jax.ShapeDtypeStruct((NC, G, 16), _F32)),
        mesh=mesh,
        scratch_types=[
            pltpu.VMEM((RPW, D), _F32),
            pltpu.VMEM((RPW,), jnp.int32),
            pltpu.VMEM((G, D), _F32),
            pltpu.VMEM((G, D), _F32),
            pltpu.VMEM((G, 16), _F32),
            pltpu.VMEM_SHARED((NS, G, D), _F32),
            pltpu.VMEM_SHARED((NS, G, D), _F32),
            pltpu.VMEM_SHARED((NS, G, 16), _F32),
            pltpu.VMEM((NS, GP, D), _F32),
            pltpu.VMEM((NS, GP, 16), _F32),
        ],
    )
    def pool(z_hbm, bi_hbm, outmax, outsum, outcnt,
             rows_v, ids_v, maxacc, sumacc, cntacc,
             stmax, stsum, stcnt, comb_v, combc_v):
        c = lax.axis_index("c")
        s = lax.axis_index("s")
        w = c * NS + s
        neg16 = jnp.full((16,), -jnp.inf, _F32)
        zero16 = jnp.zeros((16,), _F32)
        one16 = jnp.ones((16,), _F32)

        def init(i, _):
            for cc in range(D // 16):
                maxacc[i, pl.ds(cc * 16, 16)] = neg16
                sumacc[i, pl.ds(cc * 16, 16)] = zero16
            cntacc[i, :] = zero16
            return 0

        lax.fori_loop(0, G, init, 0)

        base = pl.multiple_of(w * RPW, 8)

        def process(nrows):
            pltpu.sync_copy(z_hbm.at[pl.ds(base, nrows), :],
                            rows_v.at[pl.ds(0, nrows), :])
            pltpu.sync_copy(bi_hbm.at[pl.ds(base, nrows)],
                            ids_v.at[pl.ds(0, nrows)])

            def grpbody(t, _):
                idv = ids_v[pl.ds(t * 16, 16)]
                for j in range(16):
                    g = idv[j]
                    i = t * 16 + j
                    for cc in range(D // 16):
                        v = rows_v[i, pl.ds(cc * 16, 16)]
                        m = maxacc[g, pl.ds(cc * 16, 16)]
                        maxacc[g, pl.ds(cc * 16, 16)] = jnp.maximum(m, v)
                        sv = sumacc[g, pl.ds(cc * 16, 16)]
                        sumacc[g, pl.ds(cc * 16, 16)] = sv + v
                    cntacc[g, :] = cntacc[g, :] + one16
                return 0

            lax.fori_loop(0, nrows // 16, grpbody, 0)

        @pl.when(w == NW - 1)
        def _():
            process(LAST)

        @pl.when(w != NW - 1)
        def _():
            process(RPW)

        pltpu.sync_copy(maxacc, stmax.at[s])
        pltpu.sync_copy(sumacc, stsum.at[s])
        pltpu.sync_copy(cntacc, stcnt.at[s])
        plsc.subcore_barrier()

        @pl.when(s < G // GP)
        def _():
            g0 = pl.multiple_of(s * GP, 8)
            pltpu.sync_copy(stmax.at[:, pl.ds(g0, GP), :], comb_v)

            def cmax(gg, _):
                for cc in range(D // 16):
                    m = comb_v[0, gg, pl.ds(cc * 16, 16)]
                    for t in range(1, NS):
                        m = jnp.maximum(m, comb_v[t, gg, pl.ds(cc * 16, 16)])
                    maxacc[gg, pl.ds(cc * 16, 16)] = m
                return 0

            lax.fori_loop(0, GP, cmax, 0)
            pltpu.sync_copy(maxacc.at[pl.ds(0, GP), :],
                            outmax.at[c, pl.ds(g0, GP), :])
            pltpu.sync_copy(stsum.at[:, pl.ds(g0, GP), :], comb_v)

            def csum(gg, _):
                for cc in range(D // 16):
                    m = comb_v[0, gg, pl.ds(cc * 16, 16)]
                    for t in range(1, NS):
                        m = m + comb_v[t, gg, pl.ds(cc * 16, 16)]
                    sumacc[gg, pl.ds(cc * 16, 16)] = m
                return 0

            lax.fori_loop(0, GP, csum, 0)
            pltpu.sync_copy(sumacc.at[pl.ds(0, GP), :],
                            outsum.at[c, pl.ds(g0, GP), :])
            pltpu.sync_copy(stcnt.at[:, pl.ds(g0, GP), :], combc_v)

            def ccnt(gg, _):
                m = combc_v[0, gg, :]
                for t in range(1, NS):
                    m = m + combc_v[t, gg, :]
                cntacc[gg, :] = m
                return 0

            lax.fori_loop(0, GP, ccnt, 0)
            pltpu.sync_copy(cntacc.at[pl.ds(0, GP), :],
                            outcnt.at[c, pl.ds(g0, GP), :])

    return pool


_pool = _make_pool()


_B = 2000


def _tc1_body(x_ref, degp_ref, w_ref, out_ref):
    deg = degp_ref[0, :, 0:1] + degp_ref[1, :, 0:1] + 1.0
    dinv = lax.rsqrt(deg)
    h = jnp.dot(x_ref[...], w_ref[...], preferred_element_type=_F32)
    out_ref[...] = h * dinv


def _tc1(x, degp, W1):
    return pl.pallas_call(
        _tc1_body,
        grid=(N // _B,),
        in_specs=[pl.BlockSpec((_B, D), lambda i: (i, 0)),
                  pl.BlockSpec((2, _B, 16), lambda i: (0, i, 0)),
                  pl.BlockSpec((D, D), lambda i: (0, 0))],
        out_specs=pl.BlockSpec((_B, D), lambda i: (i, 0)),
        out_shape=jax.ShapeDtypeStruct((N, D), _F32),
    )(x, degp, W1)


def _tc2_body(acc_ref, hp_ref, degp_ref, w_ref, b_ref, out_ref):
    deg = degp_ref[0, :, 0:1] + degp_ref[1, :, 0:1] + 1.0
    dinv = lax.rsqrt(deg)
    u = dinv * (acc_ref[0] + acc_ref[1] + hp_ref[...]) + b_ref[...]
    z = jnp.tanh(u)
    out_ref[...] = jnp.dot(z, w_ref[...], preferred_element_type=_F32) * dinv


def _tc2(acc, hp, degp, W2, b1):
    return pl.pallas_call(
        _tc2_body,
        grid=(N // _B,),
        in_specs=[pl.BlockSpec((2, _B, D), lambda i: (0, i, 0)),
                  pl.BlockSpec((_B, D), lambda i: (i, 0)),
                  pl.BlockSpec((2, _B, 16), lambda i: (0, i, 0)),
                  pl.BlockSpec((D, D), lambda i: (0, 0)),
                  pl.BlockSpec((1, D), lambda i: (0, 0))],
        out_specs=pl.BlockSpec((_B, D), lambda i: (i, 0)),
        out_shape=jax.ShapeDtypeStruct((N, D), _F32),
    )(acc, hp, degp, W2, b1)


def _tc3_body(acc_ref, hp_ref, degp_ref, b_ref, out_ref):
    deg = degp_ref[0, :, 0:1] + degp_ref[1, :, 0:1] + 1.0
    dinv = lax.rsqrt(deg)
    u = dinv * (acc_ref[0] + acc_ref[1] + hp_ref[...]) + b_ref[...]
    out_ref[...] = jnp.tanh(u)


def _tc3(acc, hp, degp, b2):
    return pl.pallas_call(
        _tc3_body,
        grid=(N // _B,),
        in_specs=[pl.BlockSpec((2, _B, D), lambda i: (0, i, 0)),
                  pl.BlockSpec((_B, D), lambda i: (i, 0)),
                  pl.BlockSpec((2, _B, 16), lambda i: (0, i, 0)),
                  pl.BlockSpec((1, D), lambda i: (0, 0))],
        out_specs=pl.BlockSpec((_B, D), lambda i: (i, 0)),
        out_shape=jax.ShapeDtypeStruct((N, D), _F32),
    )(acc, hp, degp, b2)


def _tc4_body(maxp_ref, sump_ref, cntp_ref, wm_ref, wv_ref, bo_ref, out_ref):
    gmax = jnp.maximum(maxp_ref[0], maxp_ref[1])
    gsum = sump_ref[0] + sump_ref[1]
    cnt = cntp_ref[0, :, 0:1] + cntp_ref[1, :, 0:1]
    gmean = gsum / jnp.maximum(cnt, 1.0)
    out = (jnp.dot(gmax, wm_ref[...], preferred_element_type=_F32)
           + jnp.dot(gmean, wv_ref[...], preferred_element_type=_F32)
           + bo_ref[...])
    out_ref[...] = out


def _tc4(maxp, sump, cntp, wm, wv, bo):
    return pl.pallas_call(
        _tc4_body,
        out_shape=jax.ShapeDtypeStruct((G, 1), _F32),
    )(maxp, sump, cntp, wm, wv, bo)


def kernel(x, edge_index, batch_index, W1, b1, W2, b2, W_out, b_out):
    src = edge_index[0]
    dst = edge_index[1]
    degp = _deg(dst)
    h1p = _tc1(x, degp, W1)
    acc1 = _msgpass(h1p, src, dst)
    h2p = _tc2(acc1, h1p, degp, W2, b1.reshape(1, D))
    acc2 = _msgpass(h2p, src, dst)
    z2 = _tc3(acc2, h2p, degp, b2.reshape(1, D))
    maxp, sump, cntp = _pool(z2, batch_index)
    out = _tc4(maxp, sump, cntp,
               W_out[:D], W_out[D:], b_out.reshape(1, 1))
    return out


# speedup vs baseline: 32.9670x; 2.5408x over previous
import functools

import jax
import jax.numpy as jnp
from jax import lax
from jax.experimental import pallas as pl
from jax.experimental.pallas import tpu as pltpu
from jax.experimental.pallas import tpu_sc as plsc

NC = 2
NS = 16
NW = NC * NS

N = 10000
E = 320000
D = 128
G = 64

_F32 = jnp.float32


def _make_deg():
    EPT = E // NW
    K = 80
    NCHUNK = EPT // K
    CO = 640
    CO_LAST = N - CO * (NS - 1)
    ZR = 80
    mesh = plsc.VectorSubcoreMesh(core_axis_name="c", subcore_axis_name="s")

    FD = 25

    @functools.partial(
        pl.kernel,
        out_type=jax.ShapeDtypeStruct((NC, N, 16), _F32),
        mesh=mesh,
        scratch_types=[
            pltpu.VMEM((NCHUNK, K), jnp.int32),
            pltpu.VMEM((K, 16), _F32),
            pltpu.VMEM((ZR, 16), _F32),
            pltpu.VMEM_SHARED((N, 16), _F32),
            pltpu.SemaphoreType.DMA,
        ],
    )
    def deg_kernel(dst_hbm, out_hbm, idx_v, ones_v, zero_v, deg_sh, sem):
        c = lax.axis_index("c")
        s = lax.axis_index("s")
        w = c * NS + s
        ones16 = jnp.ones((16,), _F32)
        zero16 = jnp.zeros((16,), _F32)

        def init_ones(i, _):
            ones_v[i, :] = ones16
            return 0

        lax.fori_loop(0, K, init_ones, 0)

        def init_zero(i, _):
            zero_v[i, :] = zero16
            return 0

        lax.fori_loop(0, ZR, init_zero, 0)
        pltpu.sync_copy(dst_hbm.at[w], idx_v)
        r0 = pl.multiple_of(s * CO, 8)
        nz = jnp.where(s == NS - 1, CO_LAST // ZR, CO // ZR)

        def zbody(k, _):
            pltpu.sync_copy(
                zero_v, deg_sh.at[pl.ds(pl.multiple_of(r0 + k * ZR, 8), ZR), :])
            return 0

        lax.fori_loop(0, nz, zbody, 0)
        plsc.subcore_barrier()

        def fire(g, _):
            pltpu.async_copy(ones_v, deg_sh.at[idx_v.at[g]], sem, add=True)
            return 0

        def drain(g, _):
            pltpu.make_async_copy(ones_v, deg_sh.at[idx_v.at[g]], sem).wait()
            return 0

        def grp(Gi, _):
            lax.fori_loop(Gi * FD, (Gi + 1) * FD, fire, 0)
            lax.fori_loop(Gi * FD, (Gi + 1) * FD, drain, 0)
            return 0

        lax.fori_loop(0, NCHUNK // FD, grp, 0)
        plsc.subcore_barrier()

        @pl.when(s < NS - 1)
        def _():
            pltpu.sync_copy(deg_sh.at[pl.ds(r0, CO), :],
                            out_hbm.at[c, pl.ds(r0, CO), :])

        @pl.when(s == NS - 1)
        def _():
            r1 = pl.multiple_of((NS - 1) * CO, 8)
            pltpu.sync_copy(deg_sh.at[pl.ds(r1, CO_LAST), :],
                            out_hbm.at[c, pl.ds(r1, CO_LAST), :])

    return deg_kernel


_deg = _make_deg()


def _make_msgpass():
    EPT = E // NW
    K = 80
    NCHUNK = EPT // K
    NBUF = 3
    NG = (NCHUNK - 2) // NBUF
    CO = 640
    CO_LAST = N - CO * (NS - 1)
    ZR = 80
    mesh = plsc.VectorSubcoreMesh(core_axis_name="c", subcore_axis_name="s")

    @functools.partial(
        pl.kernel,
        out_type=jax.ShapeDtypeStruct((NC, N, D), _F32),
        mesh=mesh,
        scratch_types=[
            pltpu.VMEM((NCHUNK, K), jnp.int32),
            [pltpu.VMEM((K,), jnp.int32) for _ in range(NBUF)],
            [pltpu.VMEM((K, D), _F32) for _ in range(NBUF)],
            pltpu.VMEM_SHARED((N, D), _F32),
            [pltpu.SemaphoreType.DMA for _ in range(NBUF)],
            [pltpu.SemaphoreType.DMA for _ in range(NBUF)],
            [pltpu.SemaphoreType.DMA for _ in range(NBUF)],
        ],
    )
    def msgpass(hp_hbm, src_hbm, dst_hbm, out_hbm,
                dst_v, sstage, rows, acc_sh, isems, gsems, sems):
        c = lax.axis_index("c")
        s = lax.axis_index("s")
        w = c * NS + s
        e0 = w * EPT
        zero16 = jnp.zeros((16,), _F32)

        def zinit(i, _):
            for cc in range(D // 16):
                rows[0][i, pl.ds(cc * 16, 16)] = zero16
            return 0

        lax.fori_loop(0, ZR, zinit, 0)
        pltpu.sync_copy(dst_hbm.at[w], dst_v)
        r0 = pl.multiple_of(s * CO, 8)
        nz = jnp.where(s == NS - 1, CO_LAST // ZR, CO // ZR)

        def zbody(k, _):
            pltpu.sync_copy(
                rows[0], acc_sh.at[pl.ds(pl.multiple_of(r0 + k * ZR, 8), ZR), :])
            return 0

        lax.fori_loop(0, nz, zbody, 0)
        plsc.subcore_barrier()

        def idx_start(g, sl):
            pltpu.async_copy(
                src_hbm.at[pl.ds(pl.multiple_of(e0 + g * K, 8), K)],
                sstage[sl], isems[sl])

        def idx_wait(g, sl):
            pltpu.make_async_copy(
                src_hbm.at[pl.ds(pl.multiple_of(e0 + g * K, 8), K)],
                sstage[sl], isems[sl]).wait()

        def gather(sl):
            pltpu.async_copy(hp_hbm.at[sstage[sl]], rows[sl], gsems[sl])

        def gather_wait(sl):
            pltpu.make_async_copy(hp_hbm.at[sstage[sl]],
                                  rows[sl], gsems[sl]).wait()

        def scatter(g, sl):
            pltpu.async_copy(rows[sl], acc_sh.at[dst_v.at[g]],
                             sems[sl], add=True)

        def scatter_drain(g, sl):
            pltpu.make_async_copy(rows[sl], acc_sh.at[dst_v.at[g]],
                                  sems[sl]).wait()

        idx_start(0, 0)
        idx_start(1, 1)
        idx_wait(0, 0)
        gather(0)

        def group(Gi, _):
            g0 = Gi * NBUF
            for j in range(NBUF):
                g = g0 + j
                nsl = (j + 1) % NBUF
                psl = (j + 2) % NBUF
                @pl.when(g >= 2)
                def _():
                    scatter_drain(g - 2, nsl)

                @pl.when(g + 2 < NCHUNK)
                def _():
                    idx_start(g + 2, psl)

                idx_wait(g + 1, nsl)
                gather(nsl)
                gather_wait(j)
                scatter(g, j)
            return 0

        lax.fori_loop(0, NG, group, 0)
        scatter_drain(NCHUNK - 4, 1)
        idx_wait(NCHUNK - 1, 1)
        gather(1)
        gather_wait(0)
        scatter(NCHUNK - 2, 0)
        gather_wait(1)
        scatter(NCHUNK - 1, 1)
        scatter_drain(NCHUNK - 3, 2)
        scatter_drain(NCHUNK - 2, 0)
        scatter_drain(NCHUNK - 1, 1)
        plsc.subcore_barrier()

        @pl.when(s < NS - 1)
        def _():
            pltpu.sync_copy(acc_sh.at[pl.ds(r0, CO), :],
                            out_hbm.at[c, pl.ds(r0, CO), :])

        @pl.when(s == NS - 1)
        def _():
            r1 = pl.multiple_of((NS - 1) * CO, 8)
            pltpu.sync_copy(acc_sh.at[pl.ds(r1, CO_LAST), :],
                            out_hbm.at[c, pl.ds(r1, CO_LAST), :])

    return msgpass


_msgpass = _make_msgpass()


def _make_pool():
    RPW = 320
    LAST = N - RPW * (NW - 1)
    GP = 8
    mesh = plsc.VectorSubcoreMesh(core_axis_name="c", subcore_axis_name="s")

    @functools.partial(
        pl.kernel,
        out_type=(jax.ShapeDtypeStruct((NC, G, D), _F32),
                  jax.ShapeDtypeStruct((NC, G, D), _F32),
                  jax.ShapeDtypeStruct((NC, G, 16), _F32)),
        mesh=mesh,
        scratch_types=[
            pltpu.VMEM((RPW, D), _F32),
            pltpu.VMEM((RPW,), jnp.int32),
            pltpu.VMEM((G, D), _F32),
            pltpu.VMEM((G, D), _F32),
            pltpu.VMEM((G, 16), _F32),
            pltpu.VMEM_SHARED((NS, G, D), _F32),
            pltpu.VMEM_SHARED((NS, G, D), _F32),
            pltpu.VMEM_SHARED((NS, G, 16), _F32),
            pltpu.VMEM((NS, GP, D), _F32),
            pltpu.VMEM((NS, GP, 16), _F32),
        ],
    )
    def pool(z_hbm, bi_hbm, outmax, outsum, outcnt,
             rows_v, ids_v, maxacc, sumacc, cntacc,
             stmax, stsum, stcnt, comb_v, combc_v):
        c = lax.axis_index("c")
        s = lax.axis_index("s")
        w = c * NS + s
        neg16 = jnp.full((16,), -jnp.inf, _F32)
        zero16 = jnp.zeros((16,), _F32)
        one16 = jnp.ones((16,), _F32)

        def init(i, _):
            for cc in range(D // 16):
                maxacc[i, pl.ds(cc * 16, 16)] = neg16
                sumacc[i, pl.ds(cc * 16, 16)] = zero16
            cntacc[i, :] = zero16
            return 0

        lax.fori_loop(0, G, init, 0)

        base = pl.multiple_of(w * RPW, 8)

        def process(nrows):
            pltpu.sync_copy(z_hbm.at[pl.ds(base, nrows), :],
                            rows_v.at[pl.ds(0, nrows), :])
            pltpu.sync_copy(bi_hbm.at[pl.ds(base, nrows)],
                            ids_v.at[pl.ds(0, nrows)])

            def grpbody(t, _):
                idv = ids_v[pl.ds(t * 16, 16)]
                for j in range(16):
                    g = idv[j]
                    i = t * 16 + j
                    for cc in range(D // 16):
                        v = rows_v[i, pl.ds(cc * 16, 16)]
                        m = maxacc[g, pl.ds(cc * 16, 16)]
                        maxacc[g, pl.ds(cc * 16, 16)] = jnp.maximum(m, v)
                        sv = sumacc[g, pl.ds(cc * 16, 16)]
                        sumacc[g, pl.ds(cc * 16, 16)] = sv + v
                    cntacc[g, :] = cntacc[g, :] + one16
                return 0

            lax.fori_loop(0, nrows // 16, grpbody, 0)

        @pl.when(w == NW - 1)
        def _():
            process(LAST)

        @pl.when(w != NW - 1)
        def _():
            process(RPW)

        pltpu.sync_copy(maxacc, stmax.at[s])
        pltpu.sync_copy(sumacc, stsum.at[s])
        pltpu.sync_copy(cntacc, stcnt.at[s])
        plsc.subcore_barrier()

        @pl.when(s < G // GP)
        def _():
            g0 = pl.multiple_of(s * GP, 8)
            pltpu.sync_copy(stmax.at[:, pl.ds(g0, GP), :], comb_v)

            def cmax(gg, _):
                for cc in range(D // 16):
                    m = comb_v[0, gg, pl.ds(cc * 16, 16)]
                    for t in range(1, NS):
                        m = jnp.maximum(m, comb_v[t, gg, pl.ds(cc * 16, 16)])
                    maxacc[gg, pl.ds(cc * 16, 16)] = m
                return 0

            lax.fori_loop(0, GP, cmax, 0)
            pltpu.sync_copy(maxacc.at[pl.ds(0, GP), :],
                            outmax.at[c, pl.ds(g0, GP), :])
            pltpu.sync_copy(stsum.at[:, pl.ds(g0, GP), :], comb_v)

            def csum(gg, _):
                for cc in range(D // 16):
                    m = comb_v[0, gg, pl.ds(cc * 16, 16)]
                    for t in range(1, NS):
                        m = m + comb_v[t, gg, pl.ds(cc * 16, 16)]
                    sumacc[gg, pl.ds(cc * 16, 16)] = m
                return 0

            lax.fori_loop(0, GP, csum, 0)
            pltpu.sync_copy(sumacc.at[pl.ds(0, GP), :],
                            outsum.at[c, pl.ds(g0, GP), :])
            pltpu.sync_copy(stcnt.at[:, pl.ds(g0, GP), :], combc_v)

            def ccnt(gg, _):
                m = combc_v[0, gg, :]
                for t in range(1, NS):
                    m = m + combc_v[t, gg, :]
                cntacc[gg, :] = m
                return 0

            lax.fori_loop(0, GP, ccnt, 0)
            pltpu.sync_copy(cntacc.at[pl.ds(0, GP), :],
                            outcnt.at[c, pl.ds(g0, GP), :])

    return pool


_pool = _make_pool()


_B = 2000


def _tc1_body(x_ref, degp_ref, w_ref, out_ref):
    deg = degp_ref[0, :, 0:1] + degp_ref[1, :, 0:1] + 1.0
    dinv = lax.rsqrt(deg)
    h = jnp.dot(x_ref[...], w_ref[...], preferred_element_type=_F32)
    out_ref[...] = h * dinv


def _tc1(x, degp, W1):
    return pl.pallas_call(
        _tc1_body,
        grid=(N // _B,),
        in_specs=[pl.BlockSpec((_B, D), lambda i: (i, 0)),
                  pl.BlockSpec((2, _B, 16), lambda i: (0, i, 0)),
                  pl.BlockSpec((D, D), lambda i: (0, 0))],
        out_specs=pl.BlockSpec((_B, D), lambda i: (i, 0)),
        out_shape=jax.ShapeDtypeStruct((N, D), _F32),
    )(x, degp, W1)


def _tc2_body(acc_ref, hp_ref, degp_ref, w_ref, b_ref, out_ref):
    deg = degp_ref[0, :, 0:1] + degp_ref[1, :, 0:1] + 1.0
    dinv = lax.rsqrt(deg)
    u = dinv * (acc_ref[0] + acc_ref[1] + hp_ref[...]) + b_ref[...]
    z = jnp.tanh(u)
    out_ref[...] = jnp.dot(z, w_ref[...], preferred_element_type=_F32) * dinv


def _tc2(acc, hp, degp, W2, b1):
    return pl.pallas_call(
        _tc2_body,
        grid=(N // _B,),
        in_specs=[pl.BlockSpec((2, _B, D), lambda i: (0, i, 0)),
                  pl.BlockSpec((_B, D), lambda i: (i, 0)),
                  pl.BlockSpec((2, _B, 16), lambda i: (0, i, 0)),
                  pl.BlockSpec((D, D), lambda i: (0, 0)),
                  pl.BlockSpec((1, D), lambda i: (0, 0))],
        out_specs=pl.BlockSpec((_B, D), lambda i: (i, 0)),
        out_shape=jax.ShapeDtypeStruct((N, D), _F32),
    )(acc, hp, degp, W2, b1)


def _tc3_body(acc_ref, hp_ref, degp_ref, b_ref, out_ref):
    deg = degp_ref[0, :, 0:1] + degp_ref[1, :, 0:1] + 1.0
    dinv = lax.rsqrt(deg)
    u = dinv * (acc_ref[0] + acc_ref[1] + hp_ref[...]) + b_ref[...]
    out_ref[...] = jnp.tanh(u)


def _tc3(acc, hp, degp, b2):
    return pl.pallas_call(
        _tc3_body,
        grid=(N // _B,),
        in_specs=[pl.BlockSpec((2, _B, D), lambda i: (0, i, 0)),
                  pl.BlockSpec((_B, D), lambda i: (i, 0)),
                  pl.BlockSpec((2, _B, 16), lambda i: (0, i, 0)),
                  pl.BlockSpec((1, D), lambda i: (0, 0))],
        out_specs=pl.BlockSpec((_B, D), lambda i: (i, 0)),
        out_shape=jax.ShapeDtypeStruct((N, D), _F32),
    )(acc, hp, degp, b2)


def _tc4_body(maxp_ref, sump_ref, cntp_ref, wm_ref, wv_ref, bo_ref, out_ref):
    gmax = jnp.maximum(maxp_ref[0], maxp_ref[1])
    gsum = sump_ref[0] + sump_ref[1]
    cnt = cntp_ref[0, :, 0:1] + cntp_ref[1, :, 0:1]
    gmean = gsum / jnp.maximum(cnt, 1.0)
    out = (jnp.dot(gmax, wm_ref[...], preferred_element_type=_F32)
           + jnp.dot(gmean, wv_ref[...], preferred_element_type=_F32)
           + bo_ref[...])
    out_ref[...] = out


def _tc4(maxp, sump, cntp, wm, wv, bo):
    return pl.pallas_call(
        _tc4_body,
        out_shape=jax.ShapeDtypeStruct((G, 1), _F32),
    )(maxp, sump, cntp, wm, wv, bo)


def kernel(x, edge_index, batch_index, W1, b1, W2, b2, W_out, b_out):
    _K = 80
    _NCH = E // NW // _K
    src = edge_index[0]
    dst = edge_index[1].reshape(NW, _NCH, _K)
    degp = _deg(dst)
    h1p = _tc1(x, degp, W1)
    acc1 = _msgpass(h1p, src, dst)
    h2p = _tc2(acc1, h1p, degp, W2, b1.reshape(1, D))
    acc2 = _msgpass(h2p, src, dst)
    z2 = _tc3(acc2, h2p, degp, b2.reshape(1, D))
    maxp, sump, cntp = _pool(z2, batch_index)
    out = _tc4(maxp, sump, cntp,
               W_out[:D], W_out[D:], b_out.reshape(1, 1))
    return out


# speedup vs baseline: 33.6028x; 1.0193x over previous
import functools

import jax
import jax.numpy as jnp
from jax import lax
from jax.experimental import pallas as pl
from jax.experimental.pallas import tpu as pltpu
from jax.experimental.pallas import tpu_sc as plsc

NC = 2
NS = 16
NW = NC * NS

N = 10000
E = 320000
D = 128
G = 64

_F32 = jnp.float32


def _make_deg():
    EPT = E // NW
    K = 80
    NCHUNK = EPT // K
    CO = 640
    CO_LAST = N - CO * (NS - 1)
    ZR = 80
    mesh = plsc.VectorSubcoreMesh(core_axis_name="c", subcore_axis_name="s")

    FD = 25

    @functools.partial(
        pl.kernel,
        out_type=jax.ShapeDtypeStruct((NC, N, 16), _F32),
        mesh=mesh,
        scratch_types=[
            pltpu.VMEM((NCHUNK, K), jnp.int32),
            pltpu.VMEM((K, 16), _F32),
            pltpu.VMEM((ZR, 16), _F32),
            pltpu.VMEM_SHARED((N, 16), _F32),
            pltpu.SemaphoreType.DMA,
            pltpu.SemaphoreType.DMA,
        ],
    )
    def deg_kernel(dst_hbm, out_hbm, idx_v, ones_v, zero_v, deg_sh, sem, zsem):
        c = lax.axis_index("c")
        s = lax.axis_index("s")
        w = c * NS + s
        ones16 = jnp.ones((16,), _F32)
        zero16 = jnp.zeros((16,), _F32)

        pltpu.async_copy(dst_hbm.at[w], idx_v, sem)

        def init_ones(i, _):
            ones_v[i, :] = ones16
            return 0

        lax.fori_loop(0, K, init_ones, 0)

        def init_zero(i, _):
            zero_v[i, :] = zero16
            return 0

        lax.fori_loop(0, ZR, init_zero, 0)
        r0 = pl.multiple_of(s * CO, 8)
        nz = jnp.where(s == NS - 1, CO_LAST // ZR, CO // ZR)

        def zfire(k, _):
            pltpu.async_copy(
                zero_v, deg_sh.at[pl.ds(pl.multiple_of(r0 + k * ZR, 8), ZR), :],
                zsem)
            return 0

        lax.fori_loop(0, nz, zfire, 0)

        def zdrain(k, _):
            pltpu.make_async_copy(
                zero_v, deg_sh.at[pl.ds(pl.multiple_of(r0 + k * ZR, 8), ZR), :],
                zsem).wait()
            return 0

        lax.fori_loop(0, nz, zdrain, 0)
        pltpu.make_async_copy(dst_hbm.at[w], idx_v, sem).wait()
        plsc.subcore_barrier()

        def fire(g, _):
            pltpu.async_copy(ones_v, deg_sh.at[idx_v.at[g]], sem, add=True)
            return 0

        def drain(g, _):
            pltpu.make_async_copy(ones_v, deg_sh.at[idx_v.at[g]], sem).wait()
            return 0

        def grp(Gi, _):
            lax.fori_loop(Gi * FD, (Gi + 1) * FD, fire, 0)
            lax.fori_loop(Gi * FD, (Gi + 1) * FD, drain, 0)
            return 0

        lax.fori_loop(0, NCHUNK // FD, grp, 0)
        plsc.subcore_barrier()

        @pl.when(s < NS - 1)
        def _():
            pltpu.sync_copy(deg_sh.at[pl.ds(r0, CO), :],
                            out_hbm.at[c, pl.ds(r0, CO), :])

        @pl.when(s == NS - 1)
        def _():
            r1 = pl.multiple_of((NS - 1) * CO, 8)
            pltpu.sync_copy(deg_sh.at[pl.ds(r1, CO_LAST), :],
                            out_hbm.at[c, pl.ds(r1, CO_LAST), :])

    return deg_kernel


_deg = _make_deg()


def _make_msgpass():
    EPT = E // NW
    K = 80
    NCHUNK = EPT // K
    NBUF = 3
    NG = (NCHUNK - 2) // NBUF
    CO = 640
    CO_LAST = N - CO * (NS - 1)
    ZR = 80
    mesh = plsc.VectorSubcoreMesh(core_axis_name="c", subcore_axis_name="s")

    @functools.partial(
        pl.kernel,
        out_type=jax.ShapeDtypeStruct((NC, N, D), _F32),
        mesh=mesh,
        scratch_types=[
            pltpu.VMEM((NCHUNK, K), jnp.int32),
            [pltpu.VMEM((K,), jnp.int32) for _ in range(NBUF)],
            [pltpu.VMEM((K, D), _F32) for _ in range(NBUF)],
            pltpu.VMEM_SHARED((N, D), _F32),
            [pltpu.SemaphoreType.DMA for _ in range(NBUF)],
            [pltpu.SemaphoreType.DMA for _ in range(NBUF)],
            [pltpu.SemaphoreType.DMA for _ in range(NBUF)],
        ],
    )
    def msgpass(hp_hbm, src_hbm, dst_hbm, out_hbm,
                dst_v, sstage, rows, acc_sh, isems, gsems, sems):
        c = lax.axis_index("c")
        s = lax.axis_index("s")
        w = c * NS + s
        e0 = w * EPT
        zero16 = jnp.zeros((16,), _F32)

        pltpu.async_copy(dst_hbm.at[w], dst_v, gsems[2])
        pltpu.async_copy(src_hbm.at[pl.ds(pl.multiple_of(e0, 8), K)],
                         sstage[0], isems[0])
        pltpu.async_copy(src_hbm.at[pl.ds(pl.multiple_of(e0 + K, 8), K)],
                         sstage[1], isems[1])

        def zinit(i, _):
            for cc in range(D // 16):
                rows[0][i, pl.ds(cc * 16, 16)] = zero16
            return 0

        lax.fori_loop(0, ZR, zinit, 0)
        r0 = pl.multiple_of(s * CO, 8)
        nz = jnp.where(s == NS - 1, CO_LAST // ZR, CO // ZR)

        def zbody(k, _):
            pltpu.sync_copy(
                rows[0], acc_sh.at[pl.ds(pl.multiple_of(r0 + k * ZR, 8), ZR), :])
            return 0

        lax.fori_loop(0, nz, zbody, 0)
        pltpu.make_async_copy(dst_hbm.at[w], dst_v, gsems[2]).wait()
        plsc.subcore_barrier()

        def idx_start(g, sl):
            pltpu.async_copy(
                src_hbm.at[pl.ds(pl.multiple_of(e0 + g * K, 8), K)],
                sstage[sl], isems[sl])

        def idx_wait(g, sl):
            pltpu.make_async_copy(
                src_hbm.at[pl.ds(pl.multiple_of(e0 + g * K, 8), K)],
                sstage[sl], isems[sl]).wait()

        def gather(sl):
            pltpu.async_copy(hp_hbm.at[sstage[sl]], rows[sl], gsems[sl])

        def gather_wait(sl):
            pltpu.make_async_copy(hp_hbm.at[sstage[sl]],
                                  rows[sl], gsems[sl]).wait()

        def scatter(g, sl):
            pltpu.async_copy(rows[sl], acc_sh.at[dst_v.at[g]],
                             sems[sl], add=True)

        def scatter_drain(g, sl):
            pltpu.make_async_copy(rows[sl], acc_sh.at[dst_v.at[g]],
                                  sems[sl]).wait()

        idx_wait(0, 0)
        gather(0)

        def group(Gi, _):
            g0 = Gi * NBUF
            for j in range(NBUF):
                g = g0 + j
                nsl = (j + 1) % NBUF
                psl = (j + 2) % NBUF
                @pl.when(g >= 2)
                def _():
                    scatter_drain(g - 2, nsl)

                @pl.when(g + 2 < NCHUNK)
                def _():
                    idx_start(g + 2, psl)

                idx_wait(g + 1, nsl)
                gather(nsl)
                gather_wait(j)
                scatter(g, j)
            return 0

        lax.fori_loop(0, NG, group, 0)
        scatter_drain(NCHUNK - 4, 1)
        idx_wait(NCHUNK - 1, 1)
        gather(1)
        gather_wait(0)
        scatter(NCHUNK - 2, 0)
        gather_wait(1)
        scatter(NCHUNK - 1, 1)
        scatter_drain(NCHUNK - 3, 2)
        scatter_drain(NCHUNK - 2, 0)
        scatter_drain(NCHUNK - 1, 1)
        plsc.subcore_barrier()

        @pl.when(s < NS - 1)
        def _():
            pltpu.sync_copy(acc_sh.at[pl.ds(r0, CO), :],
                            out_hbm.at[c, pl.ds(r0, CO), :])

        @pl.when(s == NS - 1)
        def _():
            r1 = pl.multiple_of((NS - 1) * CO, 8)
            pltpu.sync_copy(acc_sh.at[pl.ds(r1, CO_LAST), :],
                            out_hbm.at[c, pl.ds(r1, CO_LAST), :])

    return msgpass


_msgpass = _make_msgpass()


def _make_pool():
    RPW = 320
    LAST = N - RPW * (NW - 1)
    GP = 8
    mesh = plsc.VectorSubcoreMesh(core_axis_name="c", subcore_axis_name="s")

    @functools.partial(
        pl.kernel,
        out_type=(jax.ShapeDtypeStruct((NC, G, D), _F32),
                  jax.ShapeDtypeStruct((NC, G, D), _F32),
                  jax.ShapeDtypeStruct((NC, G, 16), _F32)),
        mesh=mesh,
        scratch_types=[
            pltpu.VMEM((RPW, D), _F32),
            pltpu.VMEM((RPW,), jnp.int32),
            pltpu.VMEM((G, D), _F32),
            pltpu.VMEM((G, D), _F32),
            pltpu.VMEM((G, 16), _F32),
            pltpu.VMEM_SHARED((NS, G, D), _F32),
            pltpu.VMEM_SHARED((NS, G, D), _F32),
            pltpu.VMEM_SHARED((NS, G, 16), _F32),
            pltpu.VMEM((NS, GP, D), _F32),
            pltpu.VMEM((NS, GP, 16), _F32),
        ],
    )
    def pool(z_hbm, bi_hbm, outmax, outsum, outcnt,
             rows_v, ids_v, maxacc, sumacc, cntacc,
             stmax, stsum, stcnt, comb_v, combc_v):
        c = lax.axis_index("c")
        s = lax.axis_index("s")
        w = c * NS + s
        neg16 = jnp.full((16,), -jnp.inf, _F32)
        zero16 = jnp.zeros((16,), _F32)
        one16 = jnp.ones((16,), _F32)

        def init(i, _):
            for cc in range(D // 16):
                maxacc[i, pl.ds(cc * 16, 16)] = neg16
                sumacc[i, pl.ds(cc * 16, 16)] = zero16
            cntacc[i, :] = zero16
            return 0

        lax.fori_loop(0, G, init, 0)

        base = pl.multiple_of(w * RPW, 8)

        def process(nrows):
            pltpu.sync_copy(z_hbm.at[pl.ds(base, nrows), :],
                            rows_v.at[pl.ds(0, nrows), :])
            pltpu.sync_copy(bi_hbm.at[pl.ds(base, nrows)],
                            ids_v.at[pl.ds(0, nrows)])

            def grpbody(t, _):
                idv = ids_v[pl.ds(t * 16, 16)]
                for j in range(16):
                    g = idv[j]
                    i = t * 16 + j
                    for cc in range(D // 16):
                        v = rows_v[i, pl.ds(cc * 16, 16)]
                        m = maxacc[g, pl.ds(cc * 16, 16)]
                        maxacc[g, pl.ds(cc * 16, 16)] = jnp.maximum(m, v)
                        sv = sumacc[g, pl.ds(cc * 16, 16)]
                        sumacc[g, pl.ds(cc * 16, 16)] = sv + v
                    cntacc[g, :] = cntacc[g, :] + one16
                return 0

            lax.fori_loop(0, nrows // 16, grpbody, 0)

        @pl.when(w == NW - 1)
        def _():
            process(LAST)

        @pl.when(w != NW - 1)
        def _():
            process(RPW)

        pltpu.sync_copy(maxacc, stmax.at[s])
        pltpu.sync_copy(sumacc, stsum.at[s])
        pltpu.sync_copy(cntacc, stcnt.at[s])
        plsc.subcore_barrier()

        @pl.when(s < G // GP)
        def _():
            g0 = pl.multiple_of(s * GP, 8)
            pltpu.sync_copy(stmax.at[:, pl.ds(g0, GP), :], comb_v)

            def cmax(gg, _):
                for cc in range(D // 16):
                    m = comb_v[0, gg, pl.ds(cc * 16, 16)]
                    for t in range(1, NS):
                        m = jnp.maximum(m, comb_v[t, gg, pl.ds(cc * 16, 16)])
                    maxacc[gg, pl.ds(cc * 16, 16)] = m
                return 0

            lax.fori_loop(0, GP, cmax, 0)
            pltpu.sync_copy(maxacc.at[pl.ds(0, GP), :],
                            outmax.at[c, pl.ds(g0, GP), :])
            pltpu.sync_copy(stsum.at[:, pl.ds(g0, GP), :], comb_v)

            def csum(gg, _):
                for cc in range(D // 16):
                    m = comb_v[0, gg, pl.ds(cc * 16, 16)]
                    for t in range(1, NS):
                        m = m + comb_v[t, gg, pl.ds(cc * 16, 16)]
                    sumacc[gg, pl.ds(cc * 16, 16)] = m
                return 0

            lax.fori_loop(0, GP, csum, 0)
            pltpu.sync_copy(sumacc.at[pl.ds(0, GP), :],
                            outsum.at[c, pl.ds(g0, GP), :])
            pltpu.sync_copy(stcnt.at[:, pl.ds(g0, GP), :], combc_v)

            def ccnt(gg, _):
                m = combc_v[0, gg, :]
                for t in range(1, NS):
                    m = m + combc_v[t, gg, :]
                cntacc[gg, :] = m
                return 0

            lax.fori_loop(0, GP, ccnt, 0)
            pltpu.sync_copy(cntacc.at[pl.ds(0, GP), :],
                            outcnt.at[c, pl.ds(g0, GP), :])

    return pool


_pool = _make_pool()


_B = 2000


def _tc1_body(x_ref, degp_ref, w_ref, out_ref):
    deg = degp_ref[0, :, 0:1] + degp_ref[1, :, 0:1] + 1.0
    dinv = lax.rsqrt(deg)
    h = jnp.dot(x_ref[...], w_ref[...], preferred_element_type=_F32)
    out_ref[...] = h * dinv


def _tc1(x, degp, W1):
    return pl.pallas_call(
        _tc1_body,
        grid=(N // _B,),
        in_specs=[pl.BlockSpec((_B, D), lambda i: (i, 0)),
                  pl.BlockSpec((2, _B, 16), lambda i: (0, i, 0)),
                  pl.BlockSpec((D, D), lambda i: (0, 0))],
        out_specs=pl.BlockSpec((_B, D), lambda i: (i, 0)),
        out_shape=jax.ShapeDtypeStruct((N, D), _F32),
    )(x, degp, W1)


def _tc2_body(acc_ref, hp_ref, degp_ref, w_ref, b_ref, out_ref):
    deg = degp_ref[0, :, 0:1] + degp_ref[1, :, 0:1] + 1.0
    dinv = lax.rsqrt(deg)
    u = dinv * (acc_ref[0] + acc_ref[1] + hp_ref[...]) + b_ref[...]
    z = jnp.tanh(u)
    out_ref[...] = jnp.dot(z, w_ref[...], preferred_element_type=_F32) * dinv


def _tc2(acc, hp, degp, W2, b1):
    return pl.pallas_call(
        _tc2_body,
        grid=(N // _B,),
        in_specs=[pl.BlockSpec((2, _B, D), lambda i: (0, i, 0)),
                  pl.BlockSpec((_B, D), lambda i: (i, 0)),
                  pl.BlockSpec((2, _B, 16), lambda i: (0, i, 0)),
                  pl.BlockSpec((D, D), lambda i: (0, 0)),
                  pl.BlockSpec((1, D), lambda i: (0, 0))],
        out_specs=pl.BlockSpec((_B, D), lambda i: (i, 0)),
        out_shape=jax.ShapeDtypeStruct((N, D), _F32),
    )(acc, hp, degp, W2, b1)


def _tc3_body(acc_ref, hp_ref, degp_ref, b_ref, out_ref):
    deg = degp_ref[0, :, 0:1] + degp_ref[1, :, 0:1] + 1.0
    dinv = lax.rsqrt(deg)
    u = dinv * (acc_ref[0] + acc_ref[1] + hp_ref[...]) + b_ref[...]
    out_ref[...] = jnp.tanh(u)


def _tc3(acc, hp, degp, b2):
    return pl.pallas_call(
        _tc3_body,
        grid=(N // _B,),
        in_specs=[pl.BlockSpec((2, _B, D), lambda i: (0, i, 0)),
                  pl.BlockSpec((_B, D), lambda i: (i, 0)),
                  pl.BlockSpec((2, _B, 16), lambda i: (0, i, 0)),
                  pl.BlockSpec((1, D), lambda i: (0, 0))],
        out_specs=pl.BlockSpec((_B, D), lambda i: (i, 0)),
        out_shape=jax.ShapeDtypeStruct((N, D), _F32),
    )(acc, hp, degp, b2)


def _tc4_body(maxp_ref, sump_ref, cntp_ref, wm_ref, wv_ref, bo_ref, out_ref):
    gmax = jnp.maximum(maxp_ref[0], maxp_ref[1])
    gsum = sump_ref[0] + sump_ref[1]
    cnt = cntp_ref[0, :, 0:1] + cntp_ref[1, :, 0:1]
    gmean = gsum / jnp.maximum(cnt, 1.0)
    out = (jnp.dot(gmax, wm_ref[...], preferred_element_type=_F32)
           + jnp.dot(gmean, wv_ref[...], preferred_element_type=_F32)
           + bo_ref[...])
    out_ref[...] = out


def _tc4(maxp, sump, cntp, wm, wv, bo):
    return pl.pallas_call(
        _tc4_body,
        out_shape=jax.ShapeDtypeStruct((G, 1), _F32),
    )(maxp, sump, cntp, wm, wv, bo)


def kernel(x, edge_index, batch_index, W1, b1, W2, b2, W_out, b_out):
    _K = 80
    _NCH = E // NW // _K
    src = edge_index[0]
    dst = edge_index[1].reshape(NW, _NCH, _K)
    degp = _deg(dst)
    h1p = _tc1(x, degp, W1)
    acc1 = _msgpass(h1p, src, dst)
    h2p = _tc2(acc1, h1p, degp, W2, b1.reshape(1, D))
    acc2 = _msgpass(h2p, src, dst)
    z2 = _tc3(acc2, h2p, degp, b2.reshape(1, D))
    maxp, sump, cntp = _pool(z2, batch_index)
    out = _tc4(maxp, sump, cntp,
               W_out[:D], W_out[D:], b_out.reshape(1, 1))
    return out
